# baseline (speedup 1.0000x reference)
import functools
import numpy as np
import jax
import jax.numpy as jnp
from jax import lax
from jax.experimental import pallas as pl
from jax.experimental.pallas import tpu as pltpu
from jax.experimental.pallas import tpu_sc as plsc

B, N, K, D = 16, 512, 32, 128
BH = 8
C = 64
NCHUNK = (BH * N) // C
NW = 32
CHUNKS_PER_W = NCHUNK // NW
PAIRS = C * K
LANES = 16
GROUPS = PAIRS // LANES


def _sc_scatter_body(cols_hbm, vals_hbm, m_hbm,
                     idx_v, val_v, acc_v, sems, sem_in):
    wid = lax.axis_index("s") * 2 + lax.axis_index("c")
    zeros = jnp.zeros((LANES,), jnp.float32)

    def dst(chunk):
        return m_hbm.at[chunk // (N // C),
                        pl.ds((chunk % (N // C)) * (C // 8), C // 8)]

    base = wid * CHUNKS_PER_W * PAIRS
    cin = pltpu.async_copy(
        cols_hbm.at[pl.ds(base, CHUNKS_PER_W * PAIRS)], idx_v, sem_in)
    vin = pltpu.async_copy(
        vals_hbm.at[pl.ds(base, CHUNKS_PER_W * PAIRS)], val_v, sem_in)

    def zero_blk(i, carry):
        rb = i // 8
        r = i % 8
        for buf in range(2):
            for cb in range(N // 128):
                for j in range(128 // LANES):
                    acc_v[buf, rb, cb, r, pl.ds(j * LANES, LANES)] = zeros
        return carry
    lax.fori_loop(0, C, zero_blk, 0)
    cin.wait()
    vin.wait()

    def scatter_idx(g, cols):
        row = g // (K // LANES)
        rb = jnp.full((LANES,), row // 8, jnp.int32)
        r = jnp.full((LANES,), row % 8, jnp.int32)
        return [rb, lax.shift_right_logical(cols, 7), r,
                lax.bitwise_and(cols, 127)]

    for cc in range(CHUNKS_PER_W):
        buf = cc % 2
        chunk = wid * CHUNKS_PER_W + cc
        off0 = cc * PAIRS

        if cc >= 2:
            pltpu.make_async_copy(
                acc_v.at[buf], dst(chunk - 2), sems.at[buf]).wait()
            prev0 = (cc - 2) * PAIRS
            def unzero(g, carry):
                cols = idx_v[pl.ds(prev0 + g * LANES, LANES)]
                plsc.store_scatter(acc_v.at[buf], scatter_idx(g, cols), zeros)
                return carry
            lax.fori_loop(0, GROUPS, unzero, 0)

        def group(g, carry):
            cols = idx_v[pl.ds(off0 + g * LANES, LANES)]
            vals = val_v[pl.ds(off0 + g * LANES, LANES)]
            plsc.addupdate_scatter(acc_v.at[buf], scatter_idx(g, cols), vals)
            return carry

        lax.fori_loop(0, GROUPS, group, 0)
        pltpu.async_copy(acc_v.at[buf], dst(chunk), sems.at[buf])

    for cc in range(CHUNKS_PER_W - 2, CHUNKS_PER_W):
        buf = cc % 2
        chunk = wid * CHUNKS_PER_W + cc
        pltpu.make_async_copy(
            acc_v.at[buf], dst(chunk), sems.at[buf]).wait()


def _build_m_sc(cols_flat, vals_flat):
    mesh = plsc.VectorSubcoreMesh(core_axis_name="c", subcore_axis_name="s",
                                  num_cores=2, num_subcores=16)
    k = pl.kernel(
        _sc_scatter_body,
        out_type=jax.ShapeDtypeStruct((BH, N // 8, N // 128, 8, 128),
                                      jnp.float32),
        mesh=mesh,
        scratch_types=[
            pltpu.VMEM((CHUNKS_PER_W * PAIRS,), jnp.int32),
            pltpu.VMEM((CHUNKS_PER_W * PAIRS,), jnp.float32),
            pltpu.VMEM((2, C // 8, N // 128, 8, 128), jnp.float32),
            pltpu.SemaphoreType.DMA((2,)),
            pltpu.SemaphoreType.DMA,
        ],
        compiler_params=pltpu.CompilerParams(
            needs_layout_passes=False, use_tc_tiling_on_sc=False),
    )
    return k(cols_flat, vals_flat)


def _mm_body(m_ref, h_ref, o_ref):
    m = m_ref[0]
    h1 = jnp.dot(m, h_ref[0], preferred_element_type=jnp.float32)
    o_ref[0] = jnp.dot(m, h1, preferred_element_type=jnp.float32)


def _two_layer_mm(m, h):
    return pl.pallas_call(
        _mm_body,
        grid=(BH,),
        in_specs=[
            pl.BlockSpec((1, N, N), lambda b: (b, 0, 0)),
            pl.BlockSpec((1, N, D), lambda b: (b, 0, 0)),
        ],
        out_specs=pl.BlockSpec((1, N, D), lambda b: (b, 0, 0)),
        out_shape=jax.ShapeDtypeStruct((BH, N, D), jnp.float32),
    )(m, h)


@jax.jit
def kernel(node_hidden, in_node_index, in_node_mask):
    cols_flat = in_node_index.reshape(-1)
    vals_flat = in_node_mask.reshape(-1)

    half = BH * N * K
    outs = []
    for i in range(B // BH):
        m5 = _build_m_sc(cols_flat[i * half:(i + 1) * half],
                         vals_flat[i * half:(i + 1) * half])
        m = m5.transpose(0, 1, 3, 2, 4).reshape(BH, N, N)
        outs.append(_two_layer_mm(m, node_hidden[i * BH:(i + 1) * BH]))
    return jnp.concatenate(outs, axis=0)


# speedup vs baseline: 1.2459x; 1.2459x over previous
import functools
import numpy as np
import jax
import jax.numpy as jnp
from jax import lax
from jax.experimental import pallas as pl
from jax.experimental.pallas import tpu as pltpu
from jax.experimental.pallas import tpu_sc as plsc

B, N, K, D = 16, 512, 32, 128
BH = 16
C = 64
NCHUNK = (BH * N) // C
NW = 32
CHUNKS_PER_W = NCHUNK // NW
PAIRS = C * K
LANES = 16
GROUPS = PAIRS // LANES


def _sc_scatter_body(cols_hbm, vals_hbm, m_hbm,
                     idx_v, val_v, acc_v, sems, sem_in):
    wid = lax.axis_index("s") * 2 + lax.axis_index("c")
    zeros = jnp.zeros((LANES,), jnp.float32)

    def dst(chunk):
        return m_hbm.at[chunk // (N // C),
                        pl.ds((chunk % (N // C)) * (C // 8), C // 8)]

    base = wid * CHUNKS_PER_W * PAIRS
    cin = pltpu.async_copy(
        cols_hbm.at[pl.ds(base, CHUNKS_PER_W * PAIRS)], idx_v, sem_in)
    vin = pltpu.async_copy(
        vals_hbm.at[pl.ds(base, CHUNKS_PER_W * PAIRS)], val_v, sem_in)

    @plsc.parallel_loop(0, C, unroll=4)
    def _(i):
        rb = i // 8
        r = i % 8
        for buf in range(2):
            for cb in range(N // 128):
                for j in range(128 // LANES):
                    acc_v[buf, rb, cb, r, pl.ds(j * LANES, LANES)] = zeros

    cin.wait()
    vin.wait()

    def scatter_idx(g, cols):
        row = g // (K // LANES)
        rb = jnp.full((LANES,), row // 8, jnp.int32)
        r = jnp.full((LANES,), row % 8, jnp.int32)
        return [rb, lax.shift_right_logical(cols, 7), r,
                lax.bitwise_and(cols, 127)]

    for cc in range(CHUNKS_PER_W):
        buf = cc % 2
        chunk = wid * CHUNKS_PER_W + cc
        off0 = cc * PAIRS

        if cc >= 2:
            pltpu.make_async_copy(
                acc_v.at[buf], dst(chunk - 2), sems.at[buf]).wait()
            prev0 = (cc - 2) * PAIRS

            @plsc.parallel_loop(0, GROUPS, unroll=4)
            def _(g):
                cols = idx_v[pl.ds(prev0 + g * LANES, LANES)]
                plsc.store_scatter(acc_v.at[buf], scatter_idx(g, cols), zeros)

        @plsc.parallel_loop(0, GROUPS, unroll=4)
        def _(g):
            cols = idx_v[pl.ds(off0 + g * LANES, LANES)]
            vals = val_v[pl.ds(off0 + g * LANES, LANES)]
            plsc.addupdate_scatter(acc_v.at[buf], scatter_idx(g, cols), vals)
        pltpu.async_copy(acc_v.at[buf], dst(chunk), sems.at[buf])

    for cc in range(CHUNKS_PER_W - 2, CHUNKS_PER_W):
        buf = cc % 2
        chunk = wid * CHUNKS_PER_W + cc
        pltpu.make_async_copy(
            acc_v.at[buf], dst(chunk), sems.at[buf]).wait()


def _build_m_sc(cols_flat, vals_flat):
    mesh = plsc.VectorSubcoreMesh(core_axis_name="c", subcore_axis_name="s",
                                  num_cores=2, num_subcores=16)
    k = pl.kernel(
        _sc_scatter_body,
        out_type=jax.ShapeDtypeStruct((BH, N // 8, N // 128, 8, 128),
                                      jnp.float32),
        mesh=mesh,
        scratch_types=[
            pltpu.VMEM((CHUNKS_PER_W * PAIRS,), jnp.int32),
            pltpu.VMEM((CHUNKS_PER_W * PAIRS,), jnp.float32),
            pltpu.VMEM((2, C // 8, N // 128, 8, 128), jnp.float32),
            pltpu.SemaphoreType.DMA((2,)),
            pltpu.SemaphoreType.DMA,
        ],
        compiler_params=pltpu.CompilerParams(
            needs_layout_passes=False, use_tc_tiling_on_sc=False),
    )
    return k(cols_flat, vals_flat)


def _mm_body(m_ref, h_ref, o_ref):
    m = m_ref[0]
    h1 = jnp.dot(m, h_ref[0], preferred_element_type=jnp.float32)
    o_ref[0] = jnp.dot(m, h1, preferred_element_type=jnp.float32)


def _two_layer_mm(m, h):
    return pl.pallas_call(
        _mm_body,
        grid=(BH,),
        in_specs=[
            pl.BlockSpec((1, N, N), lambda b: (b, 0, 0)),
            pl.BlockSpec((1, N, D), lambda b: (b, 0, 0)),
        ],
        out_specs=pl.BlockSpec((1, N, D), lambda b: (b, 0, 0)),
        out_shape=jax.ShapeDtypeStruct((BH, N, D), jnp.float32),
    )(m, h)


@jax.jit
def kernel(node_hidden, in_node_index, in_node_mask):
    cols_flat = in_node_index.reshape(-1)
    vals_flat = in_node_mask.reshape(-1)

    m5 = _build_m_sc(cols_flat, vals_flat)
    m = m5.transpose(0, 1, 3, 2, 4).reshape(B, N, N)
    return _two_layer_mm(m, node_hidden)


# speedup vs baseline: 1.3697x; 1.0994x over previous
import functools
import numpy as np
import jax
import jax.numpy as jnp
from jax import lax
from jax.experimental import pallas as pl
from jax.experimental.pallas import tpu as pltpu
from jax.experimental.pallas import tpu_sc as plsc

B, N, K, D = 16, 512, 32, 128
BH = 16
C = 64
NCHUNK = (BH * N) // C
NW = 32
CHUNKS_PER_W = NCHUNK // NW
PAIRS = C * K
LANES = 16
GROUPS = PAIRS // LANES


def _sc_scatter_body(cols_hbm, vals_hbm, m_hbm,
                     idx_v, val_v, acc_v, sems, sem_in):
    wid = lax.axis_index("s") * 2 + lax.axis_index("c")
    zeros = jnp.zeros((LANES,), jnp.float32)

    def dst(chunk):
        return m_hbm.at[chunk // (N // C),
                        pl.ds((chunk % (N // C)) * (C // 8), C // 8)]

    bown = wid // 2
    half = wid % 2
    cin = pltpu.async_copy(
        cols_hbm.at[bown, :, pl.ds(2 * half, 2), :, :], idx_v, sem_in)
    vin = pltpu.async_copy(
        vals_hbm.at[bown, :, pl.ds(2 * half, 2), :, :], val_v, sem_in)

    @plsc.parallel_loop(0, C, unroll=4)
    def _(i):
        rb = i // 8
        r = i % 8
        for buf in range(2):
            for cb in range(N // 128):
                for j in range(128 // LANES):
                    acc_v[buf, rb, cb, r, pl.ds(j * LANES, LANES)] = zeros

    cin.wait()
    vin.wait()

    iota = lax.iota(jnp.int32, LANES)

    def scatter_idx(s, cols):
        rowv = s * LANES + iota
        return [lax.shift_right_logical(rowv, 3),
                lax.shift_right_logical(cols, 7),
                lax.bitwise_and(rowv, 7),
                lax.bitwise_and(cols, 127)]

    for cc in range(CHUNKS_PER_W):
        buf = cc % 2
        chunk = wid * CHUNKS_PER_W + cc
        nbr = cc // 2
        ns0 = (cc % 2) * C

        if cc >= 2:
            pltpu.make_async_copy(
                acc_v.at[buf], dst(chunk - 2), sems.at[buf]).wait()
            pnbr, pns0 = (cc - 2) // 2, ((cc - 2) % 2) * C

            @plsc.parallel_loop(0, GROUPS, unroll=4)
            def _(g):
                kb = g // 32
                ks = (g // 4) % 8
                s = g % 4
                cols = idx_v[kb, pnbr, ks, pl.ds(pns0 + s * LANES, LANES)]
                plsc.store_scatter(acc_v.at[buf], scatter_idx(s, cols), zeros)

        @plsc.parallel_loop(0, GROUPS, unroll=4)
        def _(g):
            kb = g // 32
            ks = (g // 4) % 8
            s = g % 4
            cols = idx_v[kb, nbr, ks, pl.ds(ns0 + s * LANES, LANES)]
            vals = val_v[kb, nbr, ks, pl.ds(ns0 + s * LANES, LANES)]
            plsc.addupdate_scatter(acc_v.at[buf], scatter_idx(s, cols), vals)
        pltpu.async_copy(acc_v.at[buf], dst(chunk), sems.at[buf])

    for cc in range(CHUNKS_PER_W - 2, CHUNKS_PER_W):
        buf = cc % 2
        chunk = wid * CHUNKS_PER_W + cc
        pltpu.make_async_copy(
            acc_v.at[buf], dst(chunk), sems.at[buf]).wait()


def _build_m_sc(cols_flat, vals_flat):
    mesh = plsc.VectorSubcoreMesh(core_axis_name="c", subcore_axis_name="s",
                                  num_cores=2, num_subcores=16)
    k = pl.kernel(
        _sc_scatter_body,
        out_type=jax.ShapeDtypeStruct((BH, N // 8, N // 128, 8, 128),
                                      jnp.float32),
        mesh=mesh,
        scratch_types=[
            pltpu.VMEM((K // 8, 2, 8, 128), jnp.int32),
            pltpu.VMEM((K // 8, 2, 8, 128), jnp.float32),
            pltpu.VMEM((2, C // 8, N // 128, 8, 128), jnp.float32),
            pltpu.SemaphoreType.DMA((2,)),
            pltpu.SemaphoreType.DMA,
        ],
        compiler_params=pltpu.CompilerParams(
            needs_layout_passes=False, use_tc_tiling_on_sc=False),
    )
    return k(cols_flat, vals_flat)


def _mm_body(m_ref, h_ref, o_ref):
    m = m_ref[0]
    h1 = jnp.dot(m, h_ref[0], preferred_element_type=jnp.float32)
    o_ref[0] = jnp.dot(m, h1, preferred_element_type=jnp.float32)


def _two_layer_mm(m, h):
    return pl.pallas_call(
        _mm_body,
        grid=(BH,),
        in_specs=[
            pl.BlockSpec((1, N, N), lambda b: (b, 0, 0)),
            pl.BlockSpec((1, N, D), lambda b: (b, 0, 0)),
        ],
        out_specs=pl.BlockSpec((1, N, D), lambda b: (b, 0, 0)),
        out_shape=jax.ShapeDtypeStruct((BH, N, D), jnp.float32),
    )(m, h)


@jax.jit
def kernel(node_hidden, in_node_index, in_node_mask):
    def tile_view(x):
        return (x.transpose(0, 2, 1)
                 .reshape(B, K // 8, 8, N // 128, 128)
                 .transpose(0, 1, 3, 2, 4))

    cols_flat = tile_view(in_node_index)
    vals_flat = tile_view(in_node_mask)

    m5 = _build_m_sc(cols_flat, vals_flat)
    m = m5.transpose(0, 1, 3, 2, 4).reshape(B, N, N)
    return _two_layer_mm(m, node_hidden)


# speedup vs baseline: 1.5489x; 1.1308x over previous
import functools
import numpy as np
import jax
import jax.numpy as jnp
from jax import lax
from jax.experimental import pallas as pl
from jax.experimental.pallas import tpu as pltpu
from jax.experimental.pallas import tpu_sc as plsc

B, N, K, D = 16, 512, 32, 128
BH = 16
C = 64
NCHUNK = (BH * N) // C
NW = 32
CHUNKS_PER_W = NCHUNK // NW
PAIRS = C * K
LANES = 16
GROUPS = PAIRS // LANES


def _sc_scatter_body(cols_hbm, vals_hbm, m_hbm,
                     idx_v, val_v, acc_v, sems, sem_in):
    wid = lax.axis_index("s") * 2 + lax.axis_index("c")
    zeros = jnp.zeros((LANES,), jnp.float32)

    def dst(chunk):
        return m_hbm.at[chunk // (N // C),
                        pl.ds((chunk % (N // C)) * (C // 8), C // 8)]

    bown = wid // 2
    half = wid % 2
    cin = pltpu.async_copy(
        cols_hbm.at[bown, :, pl.ds(2 * half, 2), :, :], idx_v, sem_in)
    vin = pltpu.async_copy(
        vals_hbm.at[bown, :, pl.ds(2 * half, 2), :, :], val_v, sem_in)

    @plsc.parallel_loop(0, C, unroll=4)
    def _(i):
        rb = i // 8
        r = i % 8
        for buf in range(2):
            for cb in range(N // 128):
                for j in range(128 // LANES):
                    acc_v[buf, rb, cb, r, pl.ds(j * LANES, LANES)] = zeros

    cin.wait()
    vin.wait()

    iota = lax.iota(jnp.int32, LANES)

    def scatter_idx(s, cols):
        rowv = s * LANES + iota
        return [lax.shift_right_logical(rowv, 3),
                lax.shift_right_logical(cols, 7),
                lax.bitwise_and(rowv, 7),
                lax.bitwise_and(cols, 127)]

    for cc in range(CHUNKS_PER_W):
        buf = cc % 2
        chunk = wid * CHUNKS_PER_W + cc
        nbr = cc // 2
        ns0 = (cc % 2) * C

        if cc >= 2:
            pltpu.make_async_copy(
                acc_v.at[buf], dst(chunk - 2), sems.at[buf]).wait()
            pnbr, pns0 = (cc - 2) // 2, ((cc - 2) % 2) * C

            @plsc.parallel_loop(0, GROUPS, unroll=4)
            def _(g):
                kb = g // 32
                ks = (g // 4) % 8
                s = g % 4
                cols = idx_v[kb, pnbr, ks, pl.ds(pns0 + s * LANES, LANES)]
                plsc.store_scatter(acc_v.at[buf], scatter_idx(s, cols), zeros)

        @plsc.parallel_loop(0, GROUPS, unroll=4)
        def _(g):
            kb = g // 32
            ks = (g // 4) % 8
            s = g % 4
            cols = idx_v[kb, nbr, ks, pl.ds(ns0 + s * LANES, LANES)]
            vals = val_v[kb, nbr, ks, pl.ds(ns0 + s * LANES, LANES)]
            plsc.addupdate_scatter(acc_v.at[buf], scatter_idx(s, cols), vals)
        pltpu.async_copy(acc_v.at[buf], dst(chunk), sems.at[buf])

    for cc in range(CHUNKS_PER_W - 2, CHUNKS_PER_W):
        buf = cc % 2
        chunk = wid * CHUNKS_PER_W + cc
        pltpu.make_async_copy(
            acc_v.at[buf], dst(chunk), sems.at[buf]).wait()


def _build_m_sc(cols_flat, vals_flat):
    mesh = plsc.VectorSubcoreMesh(core_axis_name="c", subcore_axis_name="s",
                                  num_cores=2, num_subcores=16)
    k = pl.kernel(
        _sc_scatter_body,
        out_type=jax.ShapeDtypeStruct((BH, N // 8, N // 128, 8, 128),
                                      jnp.float32),
        mesh=mesh,
        scratch_types=[
            pltpu.VMEM((K // 8, 2, 8, 128), jnp.int32),
            pltpu.VMEM((K // 8, 2, 8, 128), jnp.float32),
            pltpu.VMEM((2, C // 8, N // 128, 8, 128), jnp.float32),
            pltpu.SemaphoreType.DMA((2,)),
            pltpu.SemaphoreType.DMA,
        ],
        compiler_params=pltpu.CompilerParams(
            needs_layout_passes=False, use_tc_tiling_on_sc=False),
    )
    return k(cols_flat, vals_flat)


def _mm_body(m_ref, h_ref, o_ref):
    for j in range(2):
        m = m_ref[j]
        h1 = jnp.dot(m, h_ref[j], preferred_element_type=jnp.float32)
        o_ref[j] = jnp.dot(m, h1, preferred_element_type=jnp.float32)


def _two_layer_mm(m, h):
    return pl.pallas_call(
        _mm_body,
        grid=(BH // 2,),
        in_specs=[
            pl.BlockSpec((2, N, N), lambda b: (b, 0, 0)),
            pl.BlockSpec((2, N, D), lambda b: (b, 0, 0)),
        ],
        out_specs=pl.BlockSpec((2, N, D), lambda b: (b, 0, 0)),
        out_shape=jax.ShapeDtypeStruct((BH, N, D), jnp.float32),
    )(m, h)


@jax.jit
def kernel(node_hidden, in_node_index, in_node_mask):
    def tile_view(x):
        return (x.transpose(0, 2, 1)
                 .reshape(B, K // 8, 8, N // 128, 128)
                 .transpose(0, 1, 3, 2, 4))

    cols_flat = tile_view(in_node_index)
    vals_flat = tile_view(in_node_mask)

    m5 = _build_m_sc(cols_flat, vals_flat)
    m = m5.transpose(0, 1, 3, 2, 4).reshape(B, N, N)
    return _two_layer_mm(m, node_hidden)


# speedup vs baseline: 1.6406x; 1.0592x over previous
import functools
import numpy as np
import jax
import jax.numpy as jnp
from jax import lax
from jax.experimental import pallas as pl
from jax.experimental.pallas import tpu as pltpu
from jax.experimental.pallas import tpu_sc as plsc

B, N, K, D = 16, 512, 32, 128
BH = 16
C = 64
NCHUNK = (BH * N) // C
NW = 32
CHUNKS_PER_W = NCHUNK // NW
PAIRS = C * K
LANES = 16
GROUPS = PAIRS // LANES


def _sc_scatter_body(cols_hbm, vals_hbm, m_hbm,
                     idx_v, val_v, acc_v, sems, sem_in):
    wid = lax.axis_index("s") * 2 + lax.axis_index("c")
    zeros = jnp.zeros((LANES,), jnp.float32)

    def dst(chunk):
        return m_hbm.at[chunk // (N // C),
                        pl.ds((chunk % (N // C)) * (C // 8), C // 8)]

    bown = wid // 2
    half = wid % 2
    cin = pltpu.async_copy(
        cols_hbm.at[bown, :, pl.ds(2 * half, 2), :, :], idx_v, sem_in)
    vin = pltpu.async_copy(
        vals_hbm.at[bown, :, pl.ds(2 * half, 2), :, :], val_v, sem_in)

    @plsc.parallel_loop(0, C, unroll=4)
    def _(i):
        rb = i // 8
        r = i % 8
        for buf in range(2):
            for cb in range(N // 128):
                for j in range(128 // LANES):
                    acc_v[buf, rb, cb, r, pl.ds(j * LANES, LANES)] = zeros

    cin.wait()
    vin.wait()

    iota = lax.iota(jnp.int32, LANES)

    def scatter_idx(s, cols):
        rowv = s * LANES + iota
        return [lax.shift_right_logical(rowv, 3),
                lax.shift_right_logical(cols, 7),
                lax.bitwise_and(rowv, 7),
                lax.bitwise_and(cols, 127)]

    for cc in range(CHUNKS_PER_W):
        buf = cc % 2
        chunk = wid * CHUNKS_PER_W + cc
        nbr = cc // 2
        ns0 = (cc % 2) * C

        if cc >= 2:
            pltpu.make_async_copy(
                acc_v.at[buf], dst(chunk - 2), sems.at[buf]).wait()
            pnbr, pns0 = (cc - 2) // 2, ((cc - 2) % 2) * C

            @plsc.parallel_loop(0, GROUPS, unroll=4)
            def _(g):
                kb = g // 32
                ks = (g // 4) % 8
                s = g % 4
                cols = idx_v[kb, pnbr, ks, pl.ds(pns0 + s * LANES, LANES)]
                plsc.store_scatter(acc_v.at[buf], scatter_idx(s, cols), zeros)

        @plsc.parallel_loop(0, GROUPS, unroll=4)
        def _(g):
            kb = g // 32
            ks = (g // 4) % 8
            s = g % 4
            cols = idx_v[kb, nbr, ks, pl.ds(ns0 + s * LANES, LANES)]
            vals = val_v[kb, nbr, ks, pl.ds(ns0 + s * LANES, LANES)]
            plsc.addupdate_scatter(acc_v.at[buf], scatter_idx(s, cols), vals)
        pltpu.async_copy(acc_v.at[buf], dst(chunk), sems.at[buf])

    for cc in range(CHUNKS_PER_W - 2, CHUNKS_PER_W):
        buf = cc % 2
        chunk = wid * CHUNKS_PER_W + cc
        pltpu.make_async_copy(
            acc_v.at[buf], dst(chunk), sems.at[buf]).wait()


def _build_m_sc(cols_flat, vals_flat):
    mesh = plsc.VectorSubcoreMesh(core_axis_name="c", subcore_axis_name="s",
                                  num_cores=2, num_subcores=16)
    k = pl.kernel(
        _sc_scatter_body,
        out_type=jax.ShapeDtypeStruct((BH, N // 8, N // 128, 8, 128),
                                      jnp.float32),
        mesh=mesh,
        scratch_types=[
            pltpu.VMEM((K // 8, 2, 8, 128), jnp.int32),
            pltpu.VMEM((K // 8, 2, 8, 128), jnp.float32),
            pltpu.VMEM((2, C // 8, N // 128, 8, 128), jnp.float32),
            pltpu.SemaphoreType.DMA((2,)),
            pltpu.SemaphoreType.DMA,
        ],
        compiler_params=pltpu.CompilerParams(
            needs_layout_passes=False, use_tc_tiling_on_sc=False),
    )
    return k(cols_flat, vals_flat)


def _mm_body(m_ref, h_ref, o_ref):
    for j in range(4):
        m = m_ref[j]
        h1 = jnp.dot(m, h_ref[j], preferred_element_type=jnp.float32)
        o_ref[j] = jnp.dot(m, h1, preferred_element_type=jnp.float32)


def _two_layer_mm(m, h):
    return pl.pallas_call(
        _mm_body,
        grid=(BH // 4,),
        in_specs=[
            pl.BlockSpec((4, N, N), lambda b: (b, 0, 0)),
            pl.BlockSpec((4, N, D), lambda b: (b, 0, 0)),
        ],
        out_specs=pl.BlockSpec((4, N, D), lambda b: (b, 0, 0)),
        out_shape=jax.ShapeDtypeStruct((BH, N, D), jnp.float32),
    )(m, h)


@jax.jit
def kernel(node_hidden, in_node_index, in_node_mask):
    def tile_view(x):
        return (x.transpose(0, 2, 1)
                 .reshape(B, K // 8, 8, N // 128, 128)
                 .transpose(0, 1, 3, 2, 4))

    cols_flat = tile_view(in_node_index)
    vals_flat = tile_view(in_node_mask)

    m5 = _build_m_sc(cols_flat, vals_flat)
    m = m5.transpose(0, 1, 3, 2, 4).reshape(B, N, N)
    return _two_layer_mm(m, node_hidden)


# speedup vs baseline: 1.6730x; 1.0198x over previous
import functools
import numpy as np
import jax
import jax.numpy as jnp
from jax import lax
from jax.experimental import pallas as pl
from jax.experimental.pallas import tpu as pltpu
from jax.experimental.pallas import tpu_sc as plsc

B, N, K, D = 16, 512, 32, 128
BH = 16
C = 64
NCHUNK = (BH * N) // C
NW = 32
CHUNKS_PER_W = NCHUNK // NW
PAIRS = C * K
LANES = 16
GROUPS = PAIRS // LANES


def _sc_scatter_body(cols_hbm, vals_hbm, m_hbm,
                     idx_v, val_v, acc_v, sems, sem_in):
    wid = lax.axis_index("s") * 2 + lax.axis_index("c")
    zeros = jnp.zeros((LANES,), jnp.float32)

    def dst(chunk):
        return m_hbm.at[chunk // (N // C),
                        pl.ds((chunk % (N // C)) * (C // 8), C // 8)]

    bown = wid // 2
    half = wid % 2
    cin = pltpu.async_copy(
        cols_hbm.at[bown, :, pl.ds(2 * half, 2), :, :], idx_v, sem_in)
    vin = pltpu.async_copy(
        vals_hbm.at[bown, :, pl.ds(2 * half, 2), :, :], val_v, sem_in)

    @plsc.parallel_loop(0, C, unroll=4)
    def _(i):
        rb = i // 8
        r = i % 8
        for buf in range(2):
            for cb in range(N // 128):
                for j in range(128 // LANES):
                    acc_v[buf, rb, cb, r, pl.ds(j * LANES, LANES)] = zeros

    cin.wait()
    vin.wait()

    iota = lax.iota(jnp.int32, LANES)

    def scatter_idx(s, cols):
        rowv = s * LANES + iota
        return [lax.shift_right_logical(rowv, 3),
                lax.shift_right_logical(cols, 7),
                lax.bitwise_and(rowv, 7),
                lax.bitwise_and(cols, 127)]

    for cc in range(CHUNKS_PER_W):
        buf = cc % 2
        chunk = wid * CHUNKS_PER_W + cc
        nbr = cc // 2
        ns0 = (cc % 2) * C

        if cc >= 2:
            pltpu.make_async_copy(
                acc_v.at[buf], dst(chunk - 2), sems.at[buf]).wait()
            pnbr, pns0 = (cc - 2) // 2, ((cc - 2) % 2) * C

            @plsc.parallel_loop(0, GROUPS, unroll=4)
            def _(g):
                kb = g // 32
                ks = (g // 4) % 8
                s = g % 4
                cols = idx_v[kb, pnbr, ks, pl.ds(pns0 + s * LANES, LANES)]
                plsc.store_scatter(acc_v.at[buf], scatter_idx(s, cols), zeros)

        @plsc.parallel_loop(0, GROUPS, unroll=4)
        def _(g):
            kb = g // 32
            ks = (g // 4) % 8
            s = g % 4
            cols = idx_v[kb, nbr, ks, pl.ds(ns0 + s * LANES, LANES)]
            vals = val_v[kb, nbr, ks, pl.ds(ns0 + s * LANES, LANES)]
            plsc.addupdate_scatter(acc_v.at[buf], scatter_idx(s, cols), vals)
        pltpu.async_copy(acc_v.at[buf], dst(chunk), sems.at[buf])

    for cc in range(CHUNKS_PER_W - 2, CHUNKS_PER_W):
        buf = cc % 2
        chunk = wid * CHUNKS_PER_W + cc
        pltpu.make_async_copy(
            acc_v.at[buf], dst(chunk), sems.at[buf]).wait()


def _build_m_sc(cols_flat, vals_flat):
    mesh = plsc.VectorSubcoreMesh(core_axis_name="c", subcore_axis_name="s",
                                  num_cores=2, num_subcores=16)
    k = pl.kernel(
        _sc_scatter_body,
        out_type=jax.ShapeDtypeStruct((BH, N // 8, N // 128, 8, 128),
                                      jnp.float32),
        mesh=mesh,
        scratch_types=[
            pltpu.VMEM((K // 8, 2, 8, 128), jnp.int32),
            pltpu.VMEM((K // 8, 2, 8, 128), jnp.float32),
            pltpu.VMEM((2, C // 8, N // 128, 8, 128), jnp.float32),
            pltpu.SemaphoreType.DMA((2,)),
            pltpu.SemaphoreType.DMA,
        ],
        compiler_params=pltpu.CompilerParams(
            needs_layout_passes=False, use_tc_tiling_on_sc=False),
    )
    return k(cols_flat, vals_flat)


def _mm_body(m_ref, h_ref, o_ref):
    for j in range(8):
        m = m_ref[j]
        h1 = jnp.dot(m, h_ref[j], preferred_element_type=jnp.float32)
        o_ref[j] = jnp.dot(m, h1, preferred_element_type=jnp.float32)


def _two_layer_mm(m, h):
    return pl.pallas_call(
        _mm_body,
        grid=(BH // 8,),
        in_specs=[
            pl.BlockSpec((8, N, N), lambda b: (b, 0, 0)),
            pl.BlockSpec((8, N, D), lambda b: (b, 0, 0)),
        ],
        out_specs=pl.BlockSpec((8, N, D), lambda b: (b, 0, 0)),
        out_shape=jax.ShapeDtypeStruct((BH, N, D), jnp.float32),
    )(m, h)


@jax.jit
def kernel(node_hidden, in_node_index, in_node_mask):
    def tile_view(x):
        return (x.transpose(0, 2, 1)
                 .reshape(B, K // 8, 8, N // 128, 128)
                 .transpose(0, 1, 3, 2, 4))

    cols_flat = tile_view(in_node_index)
    vals_flat = tile_view(in_node_mask)

    m5 = _build_m_sc(cols_flat, vals_flat)
    m = m5.transpose(0, 1, 3, 2, 4).reshape(B, N, N)
    return _two_layer_mm(m, node_hidden)
